# baseline (speedup 1.0000x reference)
import functools

import jax
import jax.numpy as jnp
from jax import lax
from jax.experimental import pallas as pl
from jax.experimental.pallas import tpu as pltpu
from jax.experimental.pallas import tpu_sc as plsc

_H = 1024
_C = 64


def _make_sc_kernel(n_tokens):
    info = plsc.get_sparse_core_info()
    nw = info.num_cores * info.num_subcores
    tpw = n_tokens // nw
    nch = tpw // _C
    mesh = plsc.VectorSubcoreMesh(core_axis_name="c", subcore_axis_name="s")

    @functools.partial(
        pl.kernel,
        mesh=mesh,
        out_type=jax.ShapeDtypeStruct((n_tokens, _H), jnp.float32),
        scratch_types=[
            pltpu.VMEM((_C,), jnp.int32),
            pltpu.VMEM((_C, _H), jnp.float32),
            pltpu.SemaphoreType.DMA,
        ],
    )
    def k(seg_hbm, table_hbm, out_hbm, idx_v, rows_v, sem):
        wid = lax.axis_index("s") * info.num_cores + lax.axis_index("c")
        base = wid * tpw

        def chunk_body(kk, carry):
            tok0 = base + kk * _C
            pltpu.sync_copy(seg_hbm.at[pl.ds(tok0, _C)], idx_v)
            pltpu.async_copy(table_hbm.at[idx_v], rows_v, sem).wait()
            pltpu.sync_copy(rows_v, out_hbm.at[pl.ds(tok0, _C)])
            return carry

        lax.fori_loop(0, nch, chunk_body, 0)

    return k


def kernel(segments, table):
    b, s = segments.shape
    n = b * s
    out = _make_sc_kernel(n)(segments.reshape(n), table)
    return out.reshape(b, s, _H)


# speedup vs baseline: 3.3670x; 3.3670x over previous
import functools

import jax
import jax.numpy as jnp
from jax import lax
from jax.experimental import pallas as pl
from jax.experimental.pallas import tpu as pltpu
from jax.experimental.pallas import tpu_sc as plsc

_H = 1024
_C = 32


def _make_sc_kernel(n_tokens):
    info = plsc.get_sparse_core_info()
    nw = info.num_cores * info.num_subcores
    tpw = n_tokens // nw
    nch = tpw // _C
    mesh = plsc.VectorSubcoreMesh(core_axis_name="c", subcore_axis_name="s")

    @functools.partial(
        pl.kernel,
        mesh=mesh,
        out_type=jax.ShapeDtypeStruct((n_tokens, _H), jnp.float32),
        scratch_types=[
            pltpu.VMEM((tpw,), jnp.int32),
            pltpu.VMEM((2 * _H,), jnp.float32),
            pltpu.VMEM((_C, _H), jnp.float32),
            pltpu.VMEM((_C, _H), jnp.float32),
            pltpu.SemaphoreType.DMA,
            pltpu.SemaphoreType.DMA,
        ],
    )
    def k(seg_hbm, table_hbm, out_hbm, idx_v, table_v, rows0, rows1,
          ss0, ss1):
        wid = lax.axis_index("s") * info.num_cores + lax.axis_index("c")
        base = wid * tpw
        pltpu.sync_copy(seg_hbm.at[pl.ds(base, tpw)], idx_v)
        pltpu.sync_copy(table_hbm, table_v)

        rows = (rows0, rows1)
        ssem = (ss0, ss1)

        def fill(kk, b):
            buf = rows[b]

            def grp(g, carry):
                seg16 = idx_v[pl.ds(kk * _C + g * 16, 16)]
                rbs = [seg16[t] * _H for t in range(16)]

                def colblk(c, cr):
                    off = c * 16
                    for t in range(16):
                        buf[g * 16 + t, pl.ds(off, 16)] = (
                            table_v[pl.ds(rbs[t] + off, 16)])
                    return cr

                lax.fori_loop(0, _H // 16, colblk, 0)
                return carry

            lax.fori_loop(0, _C // 16, grp, 0)

        def scatter(kk, b):
            return pltpu.async_copy(
                rows[b], out_hbm.at[pl.ds(base + kk * _C, _C)], ssem[b])

        pending = [None, None]
        fill(0, 0)
        for kk in range(nch):
            b = kk % 2
            pending[b] = scatter(kk, b)
            if kk + 1 < nch:
                nb = 1 - b
                if pending[nb] is not None:
                    pending[nb].wait()
                fill(kk + 1, nb)
        pending[0].wait()
        pending[1].wait()

    return k


def kernel(segments, table):
    b, s = segments.shape
    n = b * s
    out = _make_sc_kernel(n)(segments.reshape(n), table.reshape(2 * _H))
    return out.reshape(b, s, _H)
